# baseline (speedup 1.0000x reference)
import functools

import numpy as np
import jax
import jax.numpy as jnp
from jax import lax
from jax.experimental import pallas as pl
from jax.experimental.pallas import tpu as pltpu
from jax.experimental.pallas import tpu_sc as plsc

_MAX_LEN = 200
_EMB_DIM = 64
_NW = 32
_NBUF = 4
_LAG = 2
_BB = 128
_TSLAB = 8
_NSL = 4


def _make_pe_np():
    pos = np.expand_dims(np.arange(_MAX_LEN), 1)
    pe = pos / np.power(
        1000, 2 * np.expand_dims(np.arange(_EMB_DIM) // 2, 0) / _EMB_DIM
    )
    pe = pe.astype(np.float64)
    pe[:, 0::2] = np.sin(pe[:, 0::2])
    pe[:, 1::2] = np.cos(pe[:, 1::2])
    return pe.astype(np.float32)


_PET = np.repeat(_make_pe_np()[:, :, None], _BB, axis=2)


def _emb_sc(table2, xt, pet):
    n_b = xt.shape[1]
    bpw = n_b // _NW
    nbb = bpw // _BB
    n_chunks = _MAX_LEN * nbb
    n_rounds = _MAX_LEN
    n_slabs = _MAX_LEN // _TSLAB
    mesh = plsc.VectorSubcoreMesh(core_axis_name="c", subcore_axis_name="s")

    @functools.partial(
        pl.kernel,
        mesh=mesh,
        out_type=jax.ShapeDtypeStruct((_MAX_LEN, _EMB_DIM, n_b), jnp.float32),
        scratch_types=[
            pltpu.VMEM((2, _TSLAB, bpw), jnp.int32),
            pltpu.VMEM((2, _TSLAB, bpw), jnp.int32),
            pltpu.VMEM((_NBUF, _BB, 2 * _EMB_DIM), jnp.float32),
            pltpu.VMEM((_NSL, _EMB_DIM, _BB), jnp.float32),
            pltpu.SemaphoreType.DMA,
            pltpu.SemaphoreType.DMA((_NBUF,)),
            pltpu.SemaphoreType.DMA((_NSL,)),
            pltpu.SemaphoreType.DMA((_NSL,)),
        ],
        compiler_params=pltpu.CompilerParams(needs_layout_passes=False),
    )
    def k(tab_h, xt_h, pet_h, out_h, idx_v, idx2_v, rows_v, slab_v,
          sem_ix, sem_g, sem_i, sem_o):
        cid = lax.axis_index("c")
        sid = lax.axis_index("s")
        wid = sid * 2 + cid
        bw0 = wid * bpw

        row16 = lax.iota(jnp.int32, 16)
        bvec = [row16 + 16 * i for i in range(_BB // 16)]

        def stage_idx(s_blk, buf, sync):
            src = xt_h.at[pl.ds(s_blk * _TSLAB, _TSLAB), pl.ds(bw0, bpw)]
            if sync:
                pltpu.sync_copy(src, idx_v.at[buf])
            else:
                pltpu.async_copy(src, idx_v.at[buf], sem_ix)

        def wait_idx(buf):
            pltpu.make_async_copy(
                xt_h.at[pl.ds(0, _TSLAB), pl.ds(bw0, bpw)], idx_v.at[buf], sem_ix
            ).wait()

        def shift_idx(buf):
            def vec_it(i, carry):
                for tt in range(_TSLAB):
                    sl = pl.ds(i * 16, 16)
                    idx2_v[buf, tt, sl] = lax.shift_right_logical(
                        idx_v[buf, tt, sl], 1
                    )
                return carry

            lax.fori_loop(0, bpw // 16, vec_it, 0)

        def start_gather(g, buf, tt, bb):
            pltpu.async_copy(
                tab_h.at[idx2_v.at[buf, tt, pl.ds(bb * _BB, _BB)]],
                rows_v.at[g],
                sem_g.at[g],
            )

        def wait_gather(g):
            pltpu.make_async_copy(
                tab_h.at[idx2_v.at[0, 0, pl.ds(0, _BB)]],
                rows_v.at[g],
                sem_g.at[g],
            ).wait()

        def start_init(ss, t_nx):
            pltpu.async_copy(pet_h.at[t_nx], slab_v.at[ss], sem_i.at[ss])

        def wait_init(ss):
            pltpu.make_async_copy(pet_h.at[0], slab_v.at[ss], sem_i.at[ss]).wait()

        def start_store(ss, t_cd, bb_cd):
            pltpu.async_copy(
                slab_v.at[ss],
                out_h.at[t_cd, :, pl.ds(bw0 + bb_cd * _BB, _BB)],
                sem_o.at[ss],
            )

        def wait_store(ss):
            pltpu.make_async_copy(
                slab_v.at[ss], out_h.at[0, :, pl.ds(0, _BB)], sem_o.at[ss]
            ).wait()

        def transpose_pe(g, ss, buf_cd, tt_cd, bb_cd):
            hv = []
            for i in range(_BB // 16):
                rawv = idx_v[buf_cd, tt_cd, pl.ds(bb_cd * _BB + i * 16, 16)]
                hv.append(lax.rem(rawv, 2) * _EMB_DIM)

            def d_it(d, carry):
                h = carry
                for i in range(_BB // 16):
                    v = plsc.load_gather(rows_v.at[g], [bvec[i], h[i] + d])
                    plsc.addupdate(slab_v.at[ss, d, pl.ds(16 * i, 16)], v)
                return h

            lax.fori_loop(0, _EMB_DIM, d_it, tuple(hv))

        stage_idx(0, 0, True)
        stage_idx(1, 1, False)
        shift_idx(0)
        start_init(0, 0)
        start_init(1, 0)

        for b in range(_NBUF):
            start_gather(b, 0, 0, b)
            if b >= _LAG:
                cd = b - _LAG
                ss = cd % _NSL
                wait_gather(cd % _NBUF)
                wait_init(ss)
                transpose_pe(cd % _NBUF, ss, 0, 0, cd)
                start_store(ss, 0, cd)
                start_init((cd + _LAG) % _NSL, 0)

        def round_body(r, carry):
            s_blk = r // _TSLAB
            tt = lax.rem(r, _TSLAB)
            buf = lax.rem(s_blk, 2)

            @pl.when(lax.rem(r, _TSLAB) == 0)
            def _():
                wait_idx(buf)
                shift_idx(buf)

            for b in range(_NBUF):
                c = r * _NBUF + b
                start_gather(b, buf, tt, b)
                sg = (b + _LAG) % _NBUF
                bb_cd = (b + _LAG) % _NBUF
                ss = (b + _LAG) % _NSL
                if b < _LAG:
                    t_cd = r - 1
                    tt_cd = lax.rem(t_cd, _TSLAB)
                    buf_cd = lax.rem(t_cd // _TSLAB, 2)
                else:
                    t_cd = r
                    tt_cd = tt
                    buf_cd = buf
                wait_gather(sg)
                wait_init(ss)
                transpose_pe(sg, ss, buf_cd, tt_cd, bb_cd)
                start_store(ss, t_cd, bb_cd)
                ss2 = (b + 2 * _LAG) % _NSL
                wait_store(ss2)
                start_init(ss2, r)
                if b == _LAG - 1:
                    @pl.when((lax.rem(r, _TSLAB) == 0) & (s_blk < n_slabs - 1))
                    def _():
                        stage_idx(s_blk + 1, 1 - buf, False)
            return carry

        lax.fori_loop(1, n_rounds, round_body, 0)

        last_buf = (n_slabs - 1) % 2
        for e in range(_LAG):
            cd = n_chunks - _LAG + e
            sg = cd % _NBUF
            ss = cd % _NSL
            wait_gather(sg)
            wait_init(ss)
            transpose_pe(sg, ss, last_buf, _TSLAB - 1, cd % _NBUF)
            start_store(ss, _MAX_LEN - 1, cd % _NBUF)
        for ss in range(_NSL):
            wait_store(ss)

    return k(table2, xt, pet)


def kernel(x, table):
    xt = jnp.transpose(x.astype(jnp.int32))
    table2 = table.reshape(table.shape[0] // 2, 2 * _EMB_DIM)
    pet = jnp.asarray(_PET)
    out_t = _emb_sc(table2, xt, pet)
    return jnp.transpose(out_t, (2, 0, 1))


# speedup vs baseline: 1.9683x; 1.9683x over previous
import functools

import numpy as np
import jax
import jax.numpy as jnp
from jax import lax
from jax.experimental import pallas as pl
from jax.experimental.pallas import tpu as pltpu
from jax.experimental.pallas import tpu_sc as plsc

_MAX_LEN = 200
_EMB_DIM = 64
_NW = 32
_NBUF = 4
_LAG = 2
_BLKR = 16
_L0 = 104
_L1 = 96


def _make_pe_np():
    pos = np.expand_dims(np.arange(_MAX_LEN), 1)
    pe = pos / np.power(
        1000, 2 * np.expand_dims(np.arange(_EMB_DIM) // 2, 0) / _EMB_DIM
    )
    pe = pe.astype(np.float64)
    pe[:, 0::2] = np.sin(pe[:, 0::2])
    pe[:, 1::2] = np.cos(pe[:, 1::2])
    return pe.astype(np.float32)


_PE = _make_pe_np()
_HL = (_L0, _L1)
_HOFF = (0, _L0)


def _emb_sc(table2, xi, pe):
    n_rows = xi.shape[0]
    rows_per_w = n_rows // _NW
    n_chunks = rows_per_w * 2
    n_blks = rows_per_w // _BLKR
    rpb = _BLKR * 2 // _NBUF
    n_rounds = n_chunks // _NBUF
    mesh = plsc.VectorSubcoreMesh(core_axis_name="c", subcore_axis_name="s")

    @functools.partial(
        pl.kernel,
        mesh=mesh,
        out_type=jax.ShapeDtypeStruct((n_rows, _MAX_LEN, _EMB_DIM), jnp.float32),
        scratch_types=[
            pltpu.VMEM((2, _BLKR, _MAX_LEN), jnp.int32),
            pltpu.VMEM((2, _BLKR, 256), jnp.int32),
            pltpu.VMEM((_MAX_LEN, _EMB_DIM), jnp.float32),
            pltpu.VMEM((_NBUF, _L0, 2 * _EMB_DIM), jnp.float32),
            pltpu.VMEM((2, _L0, _EMB_DIM), jnp.float32),
            pltpu.SemaphoreType.DMA,
            pltpu.SemaphoreType.DMA((_NBUF,)),
            pltpu.SemaphoreType.DMA((2,)),
        ],
        compiler_params=pltpu.CompilerParams(needs_layout_passes=False),
    )
    def k(tab_h, xi_h, pe_h, out_h, idx_v, idx2_v, pe_v, rows_v,
          cbuf_v, sem_ix, sem_g, sem_o):
        cid = lax.axis_index("c")
        sid = lax.axis_index("s")
        wid = sid * 2 + cid
        base_row = wid * rows_per_w
        pltpu.sync_copy(pe_h, pe_v)

        def stage_idx(s_blk, buf, sync):
            src = xi_h.at[pl.ds(base_row + s_blk * _BLKR, _BLKR)]
            if sync:
                pltpu.sync_copy(src, idx_v.at[buf])
            else:
                pltpu.async_copy(src, idx_v.at[buf], sem_ix)

        def wait_idx(buf):
            pltpu.make_async_copy(
                xi_h.at[pl.ds(0, _BLKR)], idx_v.at[buf], sem_ix
            ).wait()

        def shift_idx(buf):
            def row_it(rr, carry):
                for h in range(2):
                    s0 = _HOFF[h]
                    d0 = h * 128
                    nv = (_HL[h] + 15) // 16
                    for v in range(nv):
                        so = min(v * 16, _HL[h] - 16)
                        idx2_v[buf, rr, pl.ds(d0 + so, 16)] = idx_v[
                            buf, rr, pl.ds(s0 + so, 16)
                        ]
                return carry

            lax.fori_loop(0, _BLKR, row_it, 0)

        def start_gather(g, buf, rr, h):
            pltpu.async_copy(
                tab_h.at[idx2_v.at[buf, rr, pl.ds(h * 128, _HL[h])]],
                rows_v.at[g, pl.ds(0, _HL[h])],
                sem_g.at[g],
            )

        def wait_gather(g, h):
            pltpu.make_async_copy(
                tab_h.at[idx2_v.at[0, 0, pl.ds(0, _HL[h])]],
                rows_v.at[g, pl.ds(0, _HL[h])],
                sem_g.at[g],
            ).wait()

        def start_store(ss, row, h):
            pltpu.async_copy(
                cbuf_v.at[ss, pl.ds(0, _HL[h])],
                out_h.at[row, pl.ds(_HOFF[h], _HL[h])],
                sem_o.at[ss],
            )

        def wait_store(ss, h):
            pltpu.make_async_copy(
                cbuf_v.at[ss, pl.ds(0, _HL[h])],
                out_h.at[0, pl.ds(_HOFF[h], _HL[h])],
                sem_o.at[ss],
            ).wait()

        def compact_pe(g, ss, h):
            def row_it(i, carry):
                for j in range(4):
                    sl = pl.ds(16 * j, 16)
                    cbuf_v[ss, i, sl] = (
                        rows_v[g, i, sl] + pe_v[_HOFF[h] + i, sl]
                    )
                return carry

            lax.fori_loop(0, _HL[h], row_it, 0)

        stage_idx(0, 0, True)
        shift_idx(0)
        stage_idx(1, 1, False)

        for b in range(_NBUF):
            h = b % 2
            start_gather(b, 0, b // 2, h)
            if b >= _LAG:
                cd = b - _LAG
                wait_gather(cd % _NBUF, h)
                compact_pe(cd % _NBUF, h, h)
                start_store(h, base_row + cd // 2, h)

        def round_body(r, carry):
            s_blk = r // rpb
            buf = lax.rem(s_blk, 2)

            @pl.when(lax.rem(r, rpb) == 0)
            def _():
                wait_idx(buf)
                shift_idx(buf)

            for b in range(_NBUF):
                c = r * _NBUF + b
                h = b % 2
                row = 2 * r + b // 2
                rr = lax.rem(row, _BLKR)
                start_gather(b, buf, rr, h)
                cd_row = 2 * r + (b - 2) // 2
                sg = (b + _LAG) % _NBUF
                wait_gather(sg, h)
                wait_store(h, h)
                compact_pe(sg, h, h)
                start_store(h, base_row + cd_row, h)
                if b == _LAG - 1:
                    @pl.when((lax.rem(r, rpb) == 0) & (s_blk < n_blks - 1))
                    def _():
                        stage_idx(s_blk + 1, 1 - buf, False)
            return carry

        lax.fori_loop(1, n_rounds, round_body, 0)

        for e in range(_LAG):
            cd = n_chunks - _LAG + e
            h = cd % 2
            sg = cd % _NBUF
            wait_gather(sg, h)
            wait_store(h, h)
            compact_pe(sg, h, h)
            start_store(h, base_row + rows_per_w - 1, h)
        for h in range(2):
            wait_store(h, h)

    return k(table2, xi, pe)


def kernel(x, table):
    xi = x.astype(jnp.int32)
    table2 = jnp.pad(table, ((0, 0), (0, _EMB_DIM)))
    pe = jnp.asarray(_PE)
    return _emb_sc(table2, xi, pe)
